# Initial kernel scaffold; baseline (speedup 1.0000x reference)
import jax
import jax.numpy as jnp
from jax.experimental import pallas as pl


def kernel(edge_index, edge_values, feats, W1_w, W1_b, W2_w, W2_b):
    raise NotImplementedError("write your pallas kernel here")


# speedup vs baseline: 4.5425x; 4.5425x over previous
import functools

import jax
import jax.numpy as jnp
from jax import lax
from jax.experimental import pallas as pl
from jax.experimental.pallas import tpu as pltpu
from jax.experimental.pallas import tpu_sc as plsc

N = 10000
E = 320000
D = 128

NC = 2
NS = 16
NW = NC * NS
EPW = E // NW
CHUNK = 80
NCHUNK = EPW // CHUNK
ROWS_BASE = 624
ROWS_LAST = N - 15 * ROWS_BASE


def _sc_body(src_hbm, dst_hbm, ev_hbm, feats_hbm, out_hbm,
             src_v, dst_v, ev_v, rows_v, acc, sem):
    c = lax.axis_index("c")
    s = lax.axis_index("s")
    gw = c * NS + s

    def zrow(i, _):
        for j in range(D // 16):
            rows_v[i, pl.ds(j * 16, 16)] = jnp.zeros((16,), jnp.float32)
        return 0
    lax.fori_loop(0, CHUNK, zrow, 0)
    for t in range(8):
        zk = s * 8 + t

        @pl.when(zk < NCHUNK)
        def _():
            off = pl.multiple_of(zk * CHUNK, 8)
            pltpu.sync_copy(rows_v, acc.at[pl.ds(off, CHUNK)])
    plsc.subcore_barrier()

    ebase = gw * EPW

    def chunk_body(k, _):
        base = pl.multiple_of(ebase + k * CHUNK, 8)
        pltpu.sync_copy(src_hbm.at[pl.ds(base, CHUNK)], src_v)
        pltpu.sync_copy(dst_hbm.at[pl.ds(base, CHUNK)], dst_v)
        pltpu.sync_copy(ev_hbm.at[pl.ds(base, CHUNK)], ev_v)
        pltpu.async_copy(feats_hbm.at[src_v], rows_v, sem).wait()

        def scale(g, _):
            ev16 = ev_v[pl.ds(pl.multiple_of(g * 16, 8), 16)]
            for e in range(16):
                evb = jnp.full((16,), ev16[e], jnp.float32)
                r = g * 16 + e
                for j in range(D // 16):
                    rows_v[r, pl.ds(j * 16, 16)] = (
                        rows_v[r, pl.ds(j * 16, 16)] * evb)
            return 0
        lax.fori_loop(0, CHUNK // 16, scale, 0)
        pltpu.sync_copy(rows_v, acc.at[dst_v], add=True)
        return 0
    lax.fori_loop(0, NCHUNK, chunk_body, 0)
    plsc.subcore_barrier()

    @pl.when(s < 15)
    def _():
        off = pl.multiple_of(s * ROWS_BASE, 8)
        pltpu.sync_copy(acc.at[pl.ds(off, ROWS_BASE)],
                        out_hbm.at[c, pl.ds(off, ROWS_BASE)])

    @pl.when(s == 15)
    def _():
        off = 15 * ROWS_BASE
        pltpu.sync_copy(acc.at[pl.ds(off, ROWS_LAST)],
                        out_hbm.at[c, pl.ds(off, ROWS_LAST)])


_sc_segment = functools.partial(
    pl.kernel,
    out_type=jax.ShapeDtypeStruct((NC, N, D), jnp.float32),
    mesh=plsc.VectorSubcoreMesh(core_axis_name="c", subcore_axis_name="s"),
    scratch_types=[
        pltpu.VMEM((CHUNK,), jnp.int32),
        pltpu.VMEM((CHUNK,), jnp.int32),
        pltpu.VMEM((CHUNK,), jnp.float32),
        pltpu.VMEM((CHUNK, D), jnp.float32),
        pltpu.VMEM_SHARED((N, D), jnp.float32),
        pltpu.SemaphoreType.DMA,
    ],
)(_sc_body)


def _tc_body(lep_ref, f_ref, w1_ref, w2_ref, b1_ref, b2_ref, o_ref):
    le = lep_ref[0] + lep_ref[1]
    f = f_ref[...]
    sf = le + f
    em = le * f
    acc = lax.dot_general(sf, w1_ref[...], (((1,), (1,)), ((), ())),
                          preferred_element_type=jnp.float32)
    acc = acc + lax.dot_general(em, w2_ref[...], (((1,), (1,)), ((), ())),
                                preferred_element_type=jnp.float32)
    o_ref[...] = acc + b1_ref[...] + b2_ref[...]


_BN = 1000


def _tc_dense(lep, feats, W1_w, W1_b, W2_w, W2_b):
    return pl.pallas_call(
        _tc_body,
        grid=(N // _BN,),
        in_specs=[
            pl.BlockSpec((NC, _BN, D), lambda i: (0, i, 0)),
            pl.BlockSpec((_BN, D), lambda i: (i, 0)),
            pl.BlockSpec((D, D), lambda i: (0, 0)),
            pl.BlockSpec((D, D), lambda i: (0, 0)),
            pl.BlockSpec((1, D), lambda i: (0, 0)),
            pl.BlockSpec((1, D), lambda i: (0, 0)),
        ],
        out_specs=pl.BlockSpec((_BN, D), lambda i: (i, 0)),
        out_shape=jax.ShapeDtypeStruct((N, D), jnp.float32),
    )(lep, feats, W1_w, W2_w, W1_b.reshape(1, D), W2_b.reshape(1, D))


def kernel(edge_index, edge_values, feats, W1_w, W1_b, W2_w, W2_b):
    src = edge_index[0]
    dst = edge_index[1]
    lep = _sc_segment(src, dst, edge_values, feats)
    return _tc_dense(lep, feats, W1_w, W1_b, W2_w, W2_b)
